# baseline (speedup 1.0000x reference)
import functools

import jax
import jax.numpy as jnp
from jax import lax
from jax.experimental import pallas as pl
from jax.experimental.pallas import tpu as pltpu
from jax.experimental.pallas import tpu_sc as plsc

_LANES = 16
_SUB = 128


@functools.lru_cache(maxsize=None)
def _build(M, D, B):
  n_workers = 16
  C = B // n_workers
  NCH = C // _SUB
  assert C * n_workers == B and NCH * _SUB == C

  mesh = plsc.VectorSubcoreMesh(
      core_axis_name="c", subcore_axis_name="s", num_cores=1)
  cp = pltpu.CompilerParams(use_tc_tiling_on_sc=False)

  @functools.partial(
      pl.kernel,
      out_type=jax.ShapeDtypeStruct((M,), jnp.int32),
      mesh=mesh,
      compiler_params=cp,
      scratch_types=[
          pltpu.VMEM((NCH, _SUB), jnp.int32),
          pltpu.VMEM((NCH, _SUB), jnp.int32),
      ],
  )
  def slot_kernel(idx_hbm, slot_hbm, idx_v, pos_v):
    w = lax.axis_index("s")
    base = w * C
    for j in range(NCH):
      pltpu.sync_copy(idx_hbm.at[pl.ds(base + j * _SUB, _SUB)], idx_v.at[j])
    for j in range(NCH):
      for t in range(_SUB // _LANES):
        pos_v[j, pl.ds(t * _LANES, _LANES)] = (
            base + j * _SUB + t * _LANES + lax.iota(jnp.int32, 16))
    for j in range(NCH):
      pltpu.sync_copy(pos_v.at[j], slot_hbm.at[idx_v.at[j]])

  @functools.partial(
      pl.kernel,
      out_type=jax.ShapeDtypeStruct((B, D), jnp.float32),
      mesh=mesh,
      compiler_params=cp,
      scratch_types=[
          pltpu.VMEM((NCH, _SUB), jnp.int32),
          pltpu.VMEM((NCH, _SUB), jnp.int32),
          pltpu.VMEM((_SUB, D), jnp.float32),
          pltpu.VMEM_SHARED((B, D), jnp.float32),
      ],
  )
  def gather_kernel(mem_hbm, idx_hbm, val_hbm, slot_hbm, out_hbm,
                    idx_v, rep_v, row_v, acc_sh):
    w = lax.axis_index("s")
    base = w * C
    for j in range(NCH):
      pltpu.sync_copy(idx_hbm.at[pl.ds(base + j * _SUB, _SUB)], idx_v.at[j])
    for j in range(NCH):
      pltpu.sync_copy(slot_hbm.at[idx_v.at[j]], rep_v.at[j])
    for j in range(NCH):
      pltpu.sync_copy(mem_hbm.at[idx_v.at[j]], row_v)
      pltpu.sync_copy(row_v, acc_sh.at[rep_v.at[j]])
    plsc.subcore_barrier()
    for j in range(NCH):
      pltpu.sync_copy(val_hbm.at[pl.ds(base + j * _SUB, _SUB)], row_v)
      pltpu.sync_copy(row_v, acc_sh.at[rep_v.at[j]], add=True)
    plsc.subcore_barrier()
    for j in range(NCH):
      pltpu.sync_copy(acc_sh.at[rep_v.at[j]], row_v)
      pltpu.sync_copy(row_v, out_hbm.at[pl.ds(base + j * _SUB, _SUB)])

  def run(mem, idx, val):
    slot = slot_kernel(idx)
    return gather_kernel(mem, idx, val, slot)

  return run


def kernel(mem, idx, val):
  M, D = mem.shape
  (B,) = idx.shape
  return _build(M, D, B)(mem, idx, val)


# speedup vs baseline: 1.1498x; 1.1498x over previous
import functools

import jax
import jax.numpy as jnp
from jax import lax
from jax.experimental import pallas as pl
from jax.experimental.pallas import tpu as pltpu
from jax.experimental.pallas import tpu_sc as plsc

_LANES = 16
_SUB = 128
_GRP = 8


@functools.lru_cache(maxsize=None)
def _build(M, D, B):
  n_workers = 16
  C = B // n_workers
  NCH = C // _SUB
  assert C * n_workers == B and NCH * _SUB == C

  mesh = plsc.VectorSubcoreMesh(
      core_axis_name="c", subcore_axis_name="s", num_cores=1)
  cp_lin = pltpu.CompilerParams(use_tc_tiling_on_sc=False,
                                needs_layout_passes=False)
  cp_tiled = pltpu.CompilerParams(use_tc_tiling_on_sc=True,
                                  needs_layout_passes=False)

  @functools.partial(
      pl.kernel,
      out_type=jax.ShapeDtypeStruct((M,), jnp.int32),
      mesh=mesh,
      compiler_params=cp_lin,
      scratch_types=[
          pltpu.VMEM((NCH, _SUB), jnp.int32),
          pltpu.VMEM((NCH, _SUB), jnp.int32),
      ],
  )
  def slot_kernel(idx_hbm, slot_hbm, idx_v, pos_v):
    w = lax.axis_index("s")
    base = w * C
    for j in range(NCH):
      pltpu.sync_copy(idx_hbm.at[pl.ds(base + j * _SUB, _SUB)], idx_v.at[j])
    for j in range(NCH):
      for t in range(_SUB // _LANES):
        pos_v[j, pl.ds(t * _LANES, _LANES)] = (
            base + j * _SUB + t * _LANES + lax.iota(jnp.int32, 16))
    for j in range(NCH):
      pltpu.sync_copy(pos_v.at[j], slot_hbm.at[idx_v.at[j]])

  @functools.partial(
      pl.kernel,
      out_type=jax.ShapeDtypeStruct((B, D), jnp.float32),
      mesh=mesh,
      compiler_params=cp_tiled,
      scratch_types=[
          pltpu.VMEM((C + _LANES,), jnp.int32),
          pltpu.VMEM((_GRP, 8, D), jnp.float32),
          pltpu.VMEM((_SUB, D), jnp.float32),
          pltpu.SemaphoreType.DMA,
      ],
  )
  def fetch_kernel(mem_hbm, idx_hbm, rows_hbm, idx_v, blk_v, row_v, sem):
    w = lax.axis_index("s")
    base = w * C
    pltpu.sync_copy(idx_hbm.at[pl.ds(base, C)], idx_v.at[pl.ds(0, C)])

    def do_chunk(j, carry):
      def do_group(g, inner):
        kv = idx_v[pl.ds(j * _SUB + g * _GRP, _LANES)]
        cps = []
        for u in range(_GRP):
          kb = (kv[u] >> 3) * 8
          cps.append(
              pltpu.async_copy(mem_hbm.at[pl.ds(kb, 8), :], blk_v.at[u], sem))
        for cp in cps:
          cp.wait()
        for u in range(_GRP):
          km = kv[u] & 7
          r = g * _GRP + u
          for t in range(D // _LANES):
            row_v[r, pl.ds(t * _LANES, _LANES)] = (
                blk_v[u, km, pl.ds(t * _LANES, _LANES)])
        return inner
      lax.fori_loop(0, _SUB // _GRP, do_group, None)
      pltpu.sync_copy(row_v, rows_hbm.at[pl.ds(base + j * _SUB, _SUB)])
      return carry
    lax.fori_loop(0, NCH, do_chunk, None)

  @functools.partial(
      pl.kernel,
      out_type=jax.ShapeDtypeStruct((B, D), jnp.float32),
      mesh=mesh,
      compiler_params=cp_lin,
      scratch_types=[
          pltpu.VMEM((NCH, _SUB), jnp.int32),
          pltpu.VMEM((NCH, _SUB), jnp.int32),
          pltpu.VMEM((_SUB, D), jnp.float32),
          pltpu.VMEM_SHARED((B, D), jnp.float32),
      ],
  )
  def combine_kernel(rows_hbm, idx_hbm, val_hbm, slot_hbm, out_hbm,
                     idx_v, rep_v, row_v, acc_sh):
    w = lax.axis_index("s")
    base = w * C
    for j in range(NCH):
      pltpu.sync_copy(idx_hbm.at[pl.ds(base + j * _SUB, _SUB)], idx_v.at[j])
    for j in range(NCH):
      pltpu.sync_copy(slot_hbm.at[idx_v.at[j]], rep_v.at[j])
    for j in range(NCH):
      pltpu.sync_copy(rows_hbm.at[pl.ds(base + j * _SUB, _SUB)], row_v)
      pltpu.sync_copy(row_v, acc_sh.at[rep_v.at[j]])
    plsc.subcore_barrier()
    for j in range(NCH):
      pltpu.sync_copy(val_hbm.at[pl.ds(base + j * _SUB, _SUB)], row_v)
      pltpu.sync_copy(row_v, acc_sh.at[rep_v.at[j]], add=True)
    plsc.subcore_barrier()
    for j in range(NCH):
      pltpu.sync_copy(acc_sh.at[rep_v.at[j]], row_v)
      pltpu.sync_copy(row_v, out_hbm.at[pl.ds(base + j * _SUB, _SUB)])

  def run(mem, idx, val):
    slot = slot_kernel(idx)
    rows = fetch_kernel(mem, idx)
    return combine_kernel(rows, idx, val, slot)

  return run


def kernel(mem, idx, val):
  M, D = mem.shape
  (B,) = idx.shape
  return _build(M, D, B)(mem, idx, val)


# speedup vs baseline: 1.2937x; 1.1251x over previous
import functools

import jax
import jax.numpy as jnp
from jax import lax
from jax.experimental import pallas as pl
from jax.experimental.pallas import tpu as pltpu
from jax.experimental.pallas import tpu_sc as plsc

_LANES = 16
_SUB = 128
_GRP = 32


@functools.lru_cache(maxsize=None)
def _build(M, D, B):
  n_workers = 16
  C = B // n_workers
  NCH = C // _SUB
  assert C * n_workers == B and NCH * _SUB == C

  mesh = plsc.VectorSubcoreMesh(
      core_axis_name="c", subcore_axis_name="s", num_cores=1)
  cp_lin = pltpu.CompilerParams(use_tc_tiling_on_sc=False,
                                needs_layout_passes=False)
  cp_tiled = pltpu.CompilerParams(use_tc_tiling_on_sc=True,
                                  needs_layout_passes=False)

  @functools.partial(
      pl.kernel,
      out_type=jax.ShapeDtypeStruct((M,), jnp.int32),
      mesh=mesh,
      compiler_params=cp_lin,
      scratch_types=[
          pltpu.VMEM((NCH, _SUB), jnp.int32),
          pltpu.VMEM((NCH, _SUB), jnp.int32),
      ],
  )
  def slot_kernel(idx_hbm, slot_hbm, idx_v, pos_v):
    w = lax.axis_index("s")
    base = w * C
    for j in range(NCH):
      pltpu.sync_copy(idx_hbm.at[pl.ds(base + j * _SUB, _SUB)], idx_v.at[j])
    for j in range(NCH):
      for t in range(_SUB // _LANES):
        pos_v[j, pl.ds(t * _LANES, _LANES)] = (
            base + j * _SUB + t * _LANES + lax.iota(jnp.int32, 16))
    for j in range(NCH):
      pltpu.sync_copy(pos_v.at[j], slot_hbm.at[idx_v.at[j]])

  @functools.partial(
      pl.kernel,
      out_type=jax.ShapeDtypeStruct((B, D), jnp.float32),
      mesh=mesh,
      compiler_params=cp_tiled,
      scratch_types=[
          pltpu.VMEM((C + _LANES,), jnp.int32),
          pltpu.VMEM((_GRP, 8, D), jnp.float32),
          pltpu.VMEM((_SUB, D), jnp.float32),
          pltpu.SemaphoreType.DMA,
      ],
  )
  def fetch_kernel(mem_hbm, idx_hbm, rows_hbm, idx_v, blk_v, row_v, sem):
    w = lax.axis_index("s")
    base = w * C
    pltpu.sync_copy(idx_hbm.at[pl.ds(base, C)], idx_v.at[pl.ds(0, C)])

    def do_chunk(j, carry):
      def do_group(g, inner):
        kvs = [
            idx_v[pl.ds(j * _SUB + g * _GRP + h * _LANES, _LANES)]
            for h in range(_GRP // _LANES)
        ]
        cps = []
        for u in range(_GRP):
          kb = (kvs[u // _LANES][u % _LANES] >> 3) * 8
          cps.append(
              pltpu.async_copy(mem_hbm.at[pl.ds(kb, 8), :], blk_v.at[u], sem))
        for cp in cps:
          cp.wait()
        for u in range(_GRP):
          km = kvs[u // _LANES][u % _LANES] & 7
          r = g * _GRP + u
          for t in range(D // _LANES):
            row_v[r, pl.ds(t * _LANES, _LANES)] = (
                blk_v[u, km, pl.ds(t * _LANES, _LANES)])
        return inner
      lax.fori_loop(0, _SUB // _GRP, do_group, None)
      pltpu.sync_copy(row_v, rows_hbm.at[pl.ds(base + j * _SUB, _SUB)])
      return carry
    lax.fori_loop(0, NCH, do_chunk, None)

  @functools.partial(
      pl.kernel,
      out_type=jax.ShapeDtypeStruct((B, D), jnp.float32),
      mesh=mesh,
      compiler_params=cp_lin,
      scratch_types=[
          pltpu.VMEM((NCH, _SUB), jnp.int32),
          pltpu.VMEM((NCH, _SUB), jnp.int32),
          pltpu.VMEM((_SUB, D), jnp.float32),
          pltpu.VMEM_SHARED((B, D), jnp.float32),
      ],
  )
  def combine_kernel(rows_hbm, idx_hbm, val_hbm, slot_hbm, out_hbm,
                     idx_v, rep_v, row_v, acc_sh):
    w = lax.axis_index("s")
    base = w * C
    for j in range(NCH):
      pltpu.sync_copy(idx_hbm.at[pl.ds(base + j * _SUB, _SUB)], idx_v.at[j])
    for j in range(NCH):
      pltpu.sync_copy(slot_hbm.at[idx_v.at[j]], rep_v.at[j])
    for j in range(NCH):
      pltpu.sync_copy(rows_hbm.at[pl.ds(base + j * _SUB, _SUB)], row_v)
      pltpu.sync_copy(row_v, acc_sh.at[rep_v.at[j]])
    plsc.subcore_barrier()
    for j in range(NCH):
      pltpu.sync_copy(val_hbm.at[pl.ds(base + j * _SUB, _SUB)], row_v)
      pltpu.sync_copy(row_v, acc_sh.at[rep_v.at[j]], add=True)
    plsc.subcore_barrier()
    for j in range(NCH):
      pltpu.sync_copy(acc_sh.at[rep_v.at[j]], row_v)
      pltpu.sync_copy(row_v, out_hbm.at[pl.ds(base + j * _SUB, _SUB)])

  def run(mem, idx, val):
    slot = slot_kernel(idx)
    rows = fetch_kernel(mem, idx)
    return combine_kernel(rows, idx, val, slot)

  return run


def kernel(mem, idx, val):
  M, D = mem.shape
  (B,) = idx.shape
  return _build(M, D, B)(mem, idx, val)
